# Initial kernel scaffold; baseline (speedup 1.0000x reference)
import jax
import jax.numpy as jnp
from jax import lax
from jax.experimental import pallas as pl
from jax.experimental.pallas import tpu as pltpu
from jax.experimental.pallas import tpu_sc as plsc

_TABLE_ROWS = 1024
_DIM = 768
_B = 16 * 1024
_NW = 32
_B_PER_W = _B // _NW
_CHUNK = 128
_NCHUNK = _B_PER_W // _CHUNK


def _body(table_hbm, idx_hbm, out_hbm,
          idx_v, rows0, rows1, gsem0, gsem1, osem0, osem1):
    wid = lax.axis_index("s") * 2 + lax.axis_index("c")
    base = wid * _B_PER_W
    pltpu.sync_copy(idx_hbm.at[pl.ds(base, _B_PER_W)], idx_v)

    bufs = (rows0, rows1)
    gsems = (gsem0, gsem1)
    osems = (osem0, osem1)

    gathers = [None] * _NCHUNK
    stores = [None] * _NCHUNK

    gathers[0] = pltpu.async_copy(
        table_hbm.at[idx_v.at[pl.ds(0, _CHUNK)]], bufs[0], gsems[0])

    for c in range(_NCHUNK):
        cb = c % 2
        nb = (c + 1) % 2
        if c + 1 < _NCHUNK:
            if c >= 1:
                stores[c - 1].wait()
            gathers[c + 1] = pltpu.async_copy(
                table_hbm.at[idx_v.at[pl.ds((c + 1) * _CHUNK, _CHUNK)]],
                bufs[nb], gsems[nb])
        gathers[c].wait()
        stores[c] = pltpu.async_copy(
            bufs[cb], out_hbm.at[pl.ds(base + c * _CHUNK, _CHUNK)], osems[cb])

    stores[_NCHUNK - 1].wait()
    if _NCHUNK >= 2:
        stores[_NCHUNK - 2].wait()


@jax.jit
def _lookup(embeddings, idx_flat):
    mesh = plsc.VectorSubcoreMesh(core_axis_name="c", subcore_axis_name="s")
    return pl.kernel(
        _body,
        mesh=mesh,
        out_type=jax.ShapeDtypeStruct((_B, _DIM), jnp.float32),
        scratch_types=[
            pltpu.VMEM((_B_PER_W,), jnp.int32),
            pltpu.VMEM((_CHUNK, _DIM), jnp.float32),
            pltpu.VMEM((_CHUNK, _DIM), jnp.float32),
            pltpu.SemaphoreType.DMA,
            pltpu.SemaphoreType.DMA,
            pltpu.SemaphoreType.DMA,
            pltpu.SemaphoreType.DMA,
        ],
    )(embeddings, idx_flat)


def kernel(patch_index, embeddings):
    idx_flat = patch_index.reshape(-1)
    out = _lookup(embeddings, idx_flat)
    return out.reshape(patch_index.shape + (embeddings.shape[1],))


# speedup vs baseline: 1.7234x; 1.7234x over previous
import jax
import jax.numpy as jnp
from jax import lax
from jax.experimental import pallas as pl
from jax.experimental.pallas import tpu as pltpu
from jax.experimental.pallas import tpu_sc as plsc

_TABLE_ROWS = 1024
_DIM = 768
_B = 16 * 1024
_NW = 32
_B_PER_W = _B // _NW
_CHUNK = 64
_NCHUNK = _B_PER_W // _CHUNK


def _body(table_hbm, idx_hbm, out_hbm,
          idx_v, rows0, rows1, gsem0, gsem1, osem0, osem1):
    wid = lax.axis_index("s") * 2 + lax.axis_index("c")
    base = wid * _B_PER_W
    pltpu.sync_copy(idx_hbm.at[pl.ds(base, _B_PER_W)], idx_v)

    bufs = (rows0, rows1)
    gsems = (gsem0, gsem1)
    osems = (osem0, osem1)

    gathers = [None] * _NCHUNK
    stores = [None] * _NCHUNK

    gathers[0] = pltpu.async_copy(
        table_hbm.at[idx_v.at[pl.ds(0, _CHUNK)]], bufs[0], gsems[0])

    for c in range(_NCHUNK):
        cb = c % 2
        nb = (c + 1) % 2
        if c + 1 < _NCHUNK:
            if c >= 1:
                stores[c - 1].wait()
            gathers[c + 1] = pltpu.async_copy(
                table_hbm.at[idx_v.at[pl.ds((c + 1) * _CHUNK, _CHUNK)]],
                bufs[nb], gsems[nb])
        gathers[c].wait()
        stores[c] = pltpu.async_copy(
            bufs[cb], out_hbm.at[pl.ds(base + c * _CHUNK, _CHUNK)], osems[cb])

    stores[_NCHUNK - 1].wait()
    if _NCHUNK >= 2:
        stores[_NCHUNK - 2].wait()


@jax.jit
def _lookup(embeddings, idx_flat):
    mesh = plsc.VectorSubcoreMesh(core_axis_name="c", subcore_axis_name="s")
    return pl.kernel(
        _body,
        mesh=mesh,
        out_type=jax.ShapeDtypeStruct((_B, _DIM), jnp.float32),
        scratch_types=[
            pltpu.VMEM((_B_PER_W,), jnp.int32),
            pltpu.VMEM((_CHUNK, _DIM), jnp.float32),
            pltpu.VMEM((_CHUNK, _DIM), jnp.float32),
            pltpu.SemaphoreType.DMA,
            pltpu.SemaphoreType.DMA,
            pltpu.SemaphoreType.DMA,
            pltpu.SemaphoreType.DMA,
        ],
    )(embeddings, idx_flat)


def kernel(patch_index, embeddings):
    idx_flat = patch_index.reshape(-1)
    out = _lookup(embeddings, idx_flat)
    return out.reshape(patch_index.shape + (embeddings.shape[1],))


# speedup vs baseline: 1.7254x; 1.0011x over previous
import jax
import jax.numpy as jnp
from jax import lax
from jax.experimental import pallas as pl
from jax.experimental.pallas import tpu as pltpu
from jax.experimental.pallas import tpu_sc as plsc

_TABLE_ROWS = 1024
_DIM = 768
_B = 16 * 1024
_NW = 32
_B_PER_W = _B // _NW
_CHUNK = 32
_NCHUNK = _B_PER_W // _CHUNK
_NBUF = 4


def _body(table_hbm, idx_hbm, out_hbm, idx_v, *rest):
    bufs = rest[:_NBUF]
    gsems = rest[_NBUF:2 * _NBUF]
    osems = rest[2 * _NBUF:3 * _NBUF]

    wid = lax.axis_index("s") * 2 + lax.axis_index("c")
    base = wid * _B_PER_W
    pltpu.sync_copy(idx_hbm.at[pl.ds(base, _B_PER_W)], idx_v)

    gathers = [None] * _NCHUNK
    stores = [None] * _NCHUNK

    def start_gather(c):
        b = c % _NBUF
        gathers[c] = pltpu.async_copy(
            table_hbm.at[idx_v.at[pl.ds(c * _CHUNK, _CHUNK)]],
            bufs[b], gsems[b])

    for c in range(min(_NBUF, _NCHUNK)):
        start_gather(c)

    for c in range(_NCHUNK):
        b = c % _NBUF
        gathers[c].wait()
        stores[c] = pltpu.async_copy(
            bufs[b], out_hbm.at[pl.ds(base + c * _CHUNK, _CHUNK)], osems[b])
        nxt = c + _NBUF
        if nxt < _NCHUNK:
            stores[c].wait()
            start_gather(nxt)

    for c in range(max(0, _NCHUNK - _NBUF), _NCHUNK):
        stores[c].wait()


@jax.jit
def _lookup(embeddings, idx_flat):
    mesh = plsc.VectorSubcoreMesh(core_axis_name="c", subcore_axis_name="s")
    return pl.kernel(
        _body,
        mesh=mesh,
        out_type=jax.ShapeDtypeStruct((_B, _DIM), jnp.float32),
        scratch_types=(
            [pltpu.VMEM((_B_PER_W,), jnp.int32)]
            + [pltpu.VMEM((_CHUNK, _DIM), jnp.float32)] * _NBUF
            + [pltpu.SemaphoreType.DMA] * (2 * _NBUF)
        ),
    )(embeddings, idx_flat)


def kernel(patch_index, embeddings):
    idx_flat = patch_index.reshape(-1)
    out = _lookup(embeddings, idx_flat)
    return out.reshape(patch_index.shape + (embeddings.shape[1],))


# speedup vs baseline: 2.4237x; 1.4048x over previous
import functools

import jax
import jax.numpy as jnp
from jax import lax
from jax.experimental import pallas as pl
from jax.experimental.pallas import tpu as pltpu

_TABLE_ROWS = 1024
_DIM = 768
_B = 16 * 1024
_BB = 1024


def _tc_body(idx_ref, tab_ref, out_ref):
    idx_col = idx_ref[...]
    vids = lax.broadcasted_iota(jnp.int32, (_BB, _TABLE_ROWS), 1)
    onehot = (idx_col == vids).astype(jnp.bfloat16)
    out_ref[...] = lax.dot_general(
        onehot, tab_ref[...],
        (((1,), (0,)), ((), ())),
        preferred_element_type=jnp.float32)


@jax.jit
def _tc_lookup(embeddings, idx_flat):
    tab16 = embeddings.astype(jnp.bfloat16)
    idx_col = idx_flat.reshape(_B, 1)
    return pl.pallas_call(
        _tc_body,
        grid=(_B // _BB,),
        in_specs=[
            pl.BlockSpec((_BB, 1), lambda i: (i, 0)),
            pl.BlockSpec((_TABLE_ROWS, _DIM), lambda i: (0, 0)),
        ],
        out_specs=pl.BlockSpec((_BB, _DIM), lambda i: (i, 0)),
        out_shape=jax.ShapeDtypeStruct((_B, _DIM), jnp.float32),
    )(idx_col, tab16)


def kernel(patch_index, embeddings):
    idx_flat = patch_index.reshape(-1)
    out = _tc_lookup(embeddings, idx_flat)
    return out.reshape(patch_index.shape + (embeddings.shape[1],))
